# baseline (speedup 1.0000x reference)
import functools

import jax
import jax.numpy as jnp
from jax import lax
from jax.experimental import pallas as pl
from jax.experimental.pallas import tpu as pltpu
from jax.experimental.pallas import tpu_sc as plsc

_TB = 256
_T = 256
_LANES = 128
_NEG = -1e30


def _gate_body(x_ref, wg_ref, bg_ref, i1_ref, i2_ref, p1_ref, p2_ref,
               imp_ref, load_ref, xp_ref):
    g = pl.program_id(0)
    x = x_ref[...]
    d2 = x.shape[1] // 2
    xlu = lax.bitcast_convert_type(
        x[:, :d2].astype(jnp.bfloat16), jnp.uint16).astype(jnp.uint32)
    xru = lax.bitcast_convert_type(
        x[:, d2:].astype(jnp.bfloat16), jnp.uint16).astype(jnp.uint32)
    xp_ref[...] = lax.bitcast_convert_type((xru << 16) | xlu, jnp.int32)
    scores = jnp.dot(x, wg_ref[...],
                     preferred_element_type=jnp.float32) + bg_ref[...]
    m = jnp.max(scores, axis=1, keepdims=True)
    ex = jnp.exp(scores - m)
    s = jnp.sum(ex, axis=1, keepdims=True)
    probs = ex / s
    iota = lax.broadcasted_iota(jnp.int32, probs.shape, 1)
    m1 = jnp.max(probs, axis=1, keepdims=True)
    a1 = jnp.min(jnp.where(probs == m1, iota, 1 << 30), axis=1, keepdims=True)
    probs_m = jnp.where(iota == a1, -1.0, probs)
    m2 = jnp.max(probs_m, axis=1, keepdims=True)
    a2 = jnp.min(jnp.where(probs_m == m2, iota, 1 << 30), axis=1,
                 keepdims=True)
    e2 = jnp.exp(m2 - m1)
    denom = 1.0 + e2
    i1_ref[...] = a1
    i2_ref[...] = a2
    p1_ref[...] = 1.0 / denom
    p2_ref[...] = e2 / denom
    imp_blk = jnp.sum(probs, axis=0, keepdims=True)
    load_blk = jnp.sum((probs > 0).astype(jnp.float32), axis=0, keepdims=True)

    @pl.when(g == 0)
    def _():
        imp_ref[...] = imp_blk
        load_ref[...] = load_blk

    @pl.when(g != 0)
    def _():
        imp_ref[...] += imp_blk
        load_ref[...] += load_blk


def _gate(x_flat, Wg, bg):
    n, d = x_flat.shape
    e = Wg.shape[0]
    wgp = jnp.zeros((d, _LANES), jnp.float32).at[:, :e].set(Wg.T)
    bgp = jnp.full((1, _LANES), _NEG, jnp.float32).at[0, :e].set(bg)
    ntb = n // _TB
    return pl.pallas_call(
        _gate_body,
        grid=(ntb,),
        in_specs=[
            pl.BlockSpec((_TB, d), lambda g: (g, 0)),
            pl.BlockSpec((d, _LANES), lambda g: (0, 0)),
            pl.BlockSpec((1, _LANES), lambda g: (0, 0)),
        ],
        out_specs=[
            pl.BlockSpec((_TB, 1), lambda g: (g, 0)),
            pl.BlockSpec((_TB, 1), lambda g: (g, 0)),
            pl.BlockSpec((_TB, 1), lambda g: (g, 0)),
            pl.BlockSpec((_TB, 1), lambda g: (g, 0)),
            pl.BlockSpec((1, _LANES), lambda g: (0, 0)),
            pl.BlockSpec((1, _LANES), lambda g: (0, 0)),
            pl.BlockSpec((_TB, d // 2), lambda g: (g, 0)),
        ],
        out_shape=[
            jax.ShapeDtypeStruct((n, 1), jnp.int32),
            jax.ShapeDtypeStruct((n, 1), jnp.int32),
            jax.ShapeDtypeStruct((n, 1), jnp.float32),
            jax.ShapeDtypeStruct((n, 1), jnp.float32),
            jax.ShapeDtypeStruct((1, _LANES), jnp.float32),
            jax.ShapeDtypeStruct((1, _LANES), jnp.float32),
            jax.ShapeDtypeStruct((n, d // 2), jnp.int32),
        ],
    )(x_flat, wgp, bgp)


_NC = 2
_NS = 16
_NW = _NC * _NS


def _sc_gather(table, idx, ch):
    np_, d = idx.shape[0], table.shape[1]
    per_w = np_ // _NW
    n_ch = per_w // ch
    mesh = plsc.VectorSubcoreMesh(core_axis_name="core",
                                  subcore_axis_name="subcore")

    @functools.partial(
        pl.kernel,
        out_type=jax.ShapeDtypeStruct((np_, d), table.dtype),
        mesh=mesh,
        scratch_types=[
            pltpu.VMEM((per_w,), jnp.int32),
            pltpu.VMEM((ch, d), table.dtype),
            pltpu.VMEM((ch, d), table.dtype),
            pltpu.VMEM((ch, d), table.dtype),
            pltpu.SemaphoreType.DMA,
            pltpu.SemaphoreType.DMA,
            pltpu.SemaphoreType.DMA,
        ])
    def k(x_hbm, i_hbm, o_hbm, idx_v, row0_v, row1_v, row2_v,
          sem0, sem1, sem2):
        wid = lax.axis_index("subcore") * _NC + lax.axis_index("core")
        base = wid * per_w
        pltpu.sync_copy(i_hbm.at[pl.ds(base, per_w)], idx_v)
        bufs = (row0_v, row1_v, row2_v)
        sems = (sem0, sem1, sem2)
        cps = []
        for c in range(n_ch):
            b = c % 3
            cp = pltpu.async_copy(
                x_hbm.at[idx_v.at[pl.ds(c * ch, ch)]], bufs[b], sems[b])
            cps.append(cp)
            if c >= 2:
                cps[c - 2].wait()
                pltpu.sync_copy(bufs[(c - 2) % 3],
                                o_hbm.at[pl.ds(base + (c - 2) * ch, ch)])
        for c in range(max(n_ch - 2, 0), n_ch):
            cps[c].wait()
            pltpu.sync_copy(bufs[c % 3],
                            o_hbm.at[pl.ds(base + c * ch, ch)])

    return k(table, idx)


def _sc_combine(opad, pos0, pos1):
    n = pos0.shape[0]
    d = opad.shape[1]
    per_w = n // _NW
    ch = 32
    n_ch = per_w // ch
    mesh = plsc.VectorSubcoreMesh(core_axis_name="core",
                                  subcore_axis_name="subcore")

    @functools.partial(
        pl.kernel,
        out_type=jax.ShapeDtypeStruct((n, d), opad.dtype),
        mesh=mesh,
        scratch_types=[
            pltpu.VMEM((per_w,), jnp.int32),
            pltpu.VMEM((per_w,), jnp.int32),
            pltpu.VMEM((ch, d), jnp.float32),
            pltpu.VMEM((ch, d), jnp.float32),
            pltpu.SemaphoreType.DMA,
            pltpu.SemaphoreType.DMA,
        ])
    def k(x_hbm, i0_hbm, i1_hbm, o_hbm, i0_v, i1_v, row0_v, row1_v,
          sem0, sem1):
        wid = lax.axis_index("subcore") * _NC + lax.axis_index("core")
        base = wid * per_w
        pltpu.sync_copy(i0_hbm.at[pl.ds(base, per_w)], i0_v)
        pltpu.sync_copy(i1_hbm.at[pl.ds(base, per_w)], i1_v)
        for c in range(n_ch):
            sl = pl.ds(c * ch, ch)
            cp0 = pltpu.async_copy(x_hbm.at[i0_v.at[sl]], row0_v, sem0)
            cp1 = pltpu.async_copy(x_hbm.at[i1_v.at[sl]], row1_v, sem1)
            cp0.wait()
            cp1.wait()

            @pl.loop(0, ch)
            def _(r):
                for q in range(d // 16):
                    qs = pl.ds(q * 16, 16)
                    row0_v[r, qs] += row1_v[r, qs]

            pltpu.sync_copy(row0_v, o_hbm.at[pl.ds(base + c * ch, ch)])

    return k(opad, pos0, pos1)


def _cast_body(w_ref, o_ref):
    o_ref[...] = w_ref[...].astype(jnp.bfloat16)


def _cast_bf16(w):
    e, a, b = w.shape
    nb = b // 1024
    return pl.pallas_call(
        _cast_body,
        grid=(e, nb),
        in_specs=[pl.BlockSpec((1, a, 1024), lambda i, j: (i, 0, j))],
        out_specs=pl.BlockSpec((1, a, 1024), lambda i, j: (i, 0, j)),
        out_shape=jax.ShapeDtypeStruct(w.shape, jnp.bfloat16),
    )(w)


def _gmm_body(nb_ref, bs_ref, xs_ref, w1_ref, b1_ref, w2_ref, b2_ref,
              pp_ref, o_ref):
    e = pl.program_id(0)
    m = pl.program_id(1)

    @pl.when(m < nb_ref[e])
    def _():
        xst = xs_ref[...].astype(jnp.bfloat16).T
        ht = lax.dot_general(w1_ref[0], xst,
                             (((1,), (0,)), ((), ())),
                             precision=lax.Precision.DEFAULT,
                             preferred_element_type=jnp.float32)
        ht = jnp.maximum(ht + b1_ref[0], 0.0)
        ot = lax.dot_general(w2_ref[0], ht.astype(jnp.bfloat16),
                             (((1,), (0,)), ((), ())),
                             precision=lax.Precision.DEFAULT,
                             preferred_element_type=jnp.float32)
        o_ref[...] = (ot.T + b2_ref[0]) * pp_ref[...]


def _gmm(nblk_e, bstart, xs_pad, W1bf, b1, W2bf, b2, ppad):
    np_, d = xs_pad.shape
    e, h, _ = W1bf.shape
    mm = np_ // _T - e

    def _blk(g, m, nb, bs):
        return bs[g] + jnp.maximum(0, jnp.minimum(m, nb[g] - 1))

    grid_spec = pltpu.PrefetchScalarGridSpec(
        num_scalar_prefetch=2,
        grid=(e, mm),
        in_specs=[
            pl.BlockSpec((_T, d), lambda g, m, nb, bs: (_blk(g, m, nb, bs), 0)),
            pl.BlockSpec((1, h, d), lambda g, m, nb, bs: (g, 0, 0)),
            pl.BlockSpec((1, h, 1), lambda g, m, nb, bs: (g, 0, 0)),
            pl.BlockSpec((1, d, h), lambda g, m, nb, bs: (g, 0, 0)),
            pl.BlockSpec((1, 1, d), lambda g, m, nb, bs: (g, 0, 0)),
            pl.BlockSpec((_T, 1), lambda g, m, nb, bs: (_blk(g, m, nb, bs), 0)),
        ],
        out_specs=pl.BlockSpec(
            (_T, d), lambda g, m, nb, bs: (_blk(g, m, nb, bs), 0)),
    )
    return pl.pallas_call(
        _gmm_body,
        grid_spec=grid_spec,
        out_shape=jax.ShapeDtypeStruct((np_, d), jnp.float32),
        compiler_params=pltpu.CompilerParams(
            dimension_semantics=("arbitrary", "arbitrary")),
    )(nblk_e, bstart, xs_pad, W1bf, b1.reshape(e, h, 1),
      W2bf, b2.reshape(e, 1, d), ppad)


def kernel(x, Wg, bg, W1, b1, W2, b2):
    bx, sx, d = x.shape
    n = bx * sx
    e = Wg.shape[0]
    kk = 2
    x_flat = x.reshape(n, d)

    i1, i2, p1, p2, imp_sum, load_sum, xpack = _gate(x_flat, Wg, bg)

    e_flat = jnp.concatenate([i1, i2], axis=1).reshape(-1)
    probs_flat = jnp.concatenate([p1, p2], axis=1).reshape(-1)
    oh = (e_flat[:, None] == jnp.arange(e, dtype=jnp.int32)[None, :]
          ).astype(jnp.int32)
    cum = jnp.cumsum(oh, axis=0)
    rk = jnp.take_along_axis(cum, e_flat[:, None], axis=1)[:, 0] - 1
    counts = cum[-1]
    nblk_e = ((counts + _T - 1) // _T).astype(jnp.int32)
    cum_nblk = jnp.cumsum(nblk_e)
    bstart = jnp.concatenate(
        [jnp.zeros((1,), jnp.int32), cum_nblk[:-1]]).astype(jnp.int32)
    nblk = n * kk // _T + e
    pp = bstart[e_flat] * _T + rk
    np_ = nblk * _T
    gidx = jnp.zeros((np_,), jnp.int32).at[pp].set(
        (jnp.arange(n * kk, dtype=jnp.int32) // kk),
        unique_indices=True)
    ppad = jnp.zeros((np_, 1), jnp.float32).at[pp, 0].set(
        probs_flat, unique_indices=True)
    pos = pp.reshape(n, kk)

    xs_pad = _sc_gather(x_flat, gidx, 32)
    opad = _gmm(nblk_e, bstart, xs_pad, _cast_bf16(W1), b1,
                _cast_bf16(W2), b2, ppad)
    y = _sc_combine(opad, pos[:, 0], pos[:, 1])

    outputs = y.reshape(bx, sx, d)
    nf = jnp.float32(n)
    aux_loss = jnp.sum((imp_sum[0] / nf) * (load_sum[0] / nf)) * e
    return outputs, aux_loss


# speedup vs baseline: 1.0981x; 1.0981x over previous
import functools

import jax
import jax.numpy as jnp
from jax import lax
from jax.experimental import pallas as pl
from jax.experimental.pallas import tpu as pltpu
from jax.experimental.pallas import tpu_sc as plsc

_TB = 256
_T = 256
_LANES = 128
_NEG = -1e30


def _gate_body(x_ref, wg_ref, bg_ref, i1_ref, i2_ref, p1_ref, p2_ref,
               imp_ref, load_ref, xp_ref):
    g = pl.program_id(0)
    x = x_ref[...]
    d2 = x.shape[1] // 2
    xlu = lax.bitcast_convert_type(
        x[:, :d2].astype(jnp.bfloat16), jnp.uint16).astype(jnp.uint32)
    xru = lax.bitcast_convert_type(
        x[:, d2:].astype(jnp.bfloat16), jnp.uint16).astype(jnp.uint32)
    xp_ref[...] = lax.bitcast_convert_type((xru << 16) | xlu, jnp.int32)
    scores = jnp.dot(x, wg_ref[...],
                     preferred_element_type=jnp.float32) + bg_ref[...]
    m = jnp.max(scores, axis=1, keepdims=True)
    ex = jnp.exp(scores - m)
    s = jnp.sum(ex, axis=1, keepdims=True)
    probs = ex / s
    iota = lax.broadcasted_iota(jnp.int32, probs.shape, 1)
    m1 = jnp.max(probs, axis=1, keepdims=True)
    a1 = jnp.min(jnp.where(probs == m1, iota, 1 << 30), axis=1, keepdims=True)
    probs_m = jnp.where(iota == a1, -1.0, probs)
    m2 = jnp.max(probs_m, axis=1, keepdims=True)
    a2 = jnp.min(jnp.where(probs_m == m2, iota, 1 << 30), axis=1,
                 keepdims=True)
    e2 = jnp.exp(m2 - m1)
    denom = 1.0 + e2
    i1_ref[...] = a1
    i2_ref[...] = a2
    p1_ref[...] = 1.0 / denom
    p2_ref[...] = e2 / denom
    imp_blk = jnp.sum(probs, axis=0, keepdims=True)
    load_blk = jnp.sum((probs > 0).astype(jnp.float32), axis=0, keepdims=True)

    @pl.when(g == 0)
    def _():
        imp_ref[...] = imp_blk
        load_ref[...] = load_blk

    @pl.when(g != 0)
    def _():
        imp_ref[...] += imp_blk
        load_ref[...] += load_blk


def _gate(x_flat, Wg, bg):
    n, d = x_flat.shape
    e = Wg.shape[0]
    wgp = jnp.zeros((d, _LANES), jnp.float32).at[:, :e].set(Wg.T)
    bgp = jnp.full((1, _LANES), _NEG, jnp.float32).at[0, :e].set(bg)
    ntb = n // _TB
    return pl.pallas_call(
        _gate_body,
        grid=(ntb,),
        in_specs=[
            pl.BlockSpec((_TB, d), lambda g: (g, 0)),
            pl.BlockSpec((d, _LANES), lambda g: (0, 0)),
            pl.BlockSpec((1, _LANES), lambda g: (0, 0)),
        ],
        out_specs=[
            pl.BlockSpec((_TB, 1), lambda g: (g, 0)),
            pl.BlockSpec((_TB, 1), lambda g: (g, 0)),
            pl.BlockSpec((_TB, 1), lambda g: (g, 0)),
            pl.BlockSpec((_TB, 1), lambda g: (g, 0)),
            pl.BlockSpec((1, _LANES), lambda g: (0, 0)),
            pl.BlockSpec((1, _LANES), lambda g: (0, 0)),
            pl.BlockSpec((_TB, d // 2), lambda g: (g, 0)),
        ],
        out_shape=[
            jax.ShapeDtypeStruct((n, 1), jnp.int32),
            jax.ShapeDtypeStruct((n, 1), jnp.int32),
            jax.ShapeDtypeStruct((n, 1), jnp.float32),
            jax.ShapeDtypeStruct((n, 1), jnp.float32),
            jax.ShapeDtypeStruct((1, _LANES), jnp.float32),
            jax.ShapeDtypeStruct((1, _LANES), jnp.float32),
            jax.ShapeDtypeStruct((n, d // 2), jnp.int32),
        ],
    )(x_flat, wgp, bgp)


_NC = 2
_NS = 16
_NW = _NC * _NS


def _sc_gather(table, idx, ch):
    np_, d = idx.shape[0], table.shape[1]
    per_w = np_ // _NW
    n_ch = per_w // ch
    mesh = plsc.VectorSubcoreMesh(core_axis_name="core",
                                  subcore_axis_name="subcore")

    @functools.partial(
        pl.kernel,
        out_type=jax.ShapeDtypeStruct((np_, d), table.dtype),
        mesh=mesh,
        scratch_types=[
            pltpu.VMEM((per_w,), jnp.int32),
            pltpu.VMEM((ch, d), table.dtype),
            pltpu.VMEM((ch, d), table.dtype),
            pltpu.VMEM((ch, d), table.dtype),
            pltpu.SemaphoreType.DMA,
            pltpu.SemaphoreType.DMA,
            pltpu.SemaphoreType.DMA,
        ])
    def k(x_hbm, i_hbm, o_hbm, idx_v, row0_v, row1_v, row2_v,
          sem0, sem1, sem2):
        wid = lax.axis_index("subcore") * _NC + lax.axis_index("core")
        base = wid * per_w
        pltpu.sync_copy(i_hbm.at[pl.ds(base, per_w)], idx_v)
        bufs = (row0_v, row1_v, row2_v)
        sems = (sem0, sem1, sem2)
        cps = []
        for c in range(n_ch):
            b = c % 3
            cp = pltpu.async_copy(
                x_hbm.at[idx_v.at[pl.ds(c * ch, ch)]], bufs[b], sems[b])
            cps.append(cp)
            if c >= 2:
                cps[c - 2].wait()
                pltpu.sync_copy(bufs[(c - 2) % 3],
                                o_hbm.at[pl.ds(base + (c - 2) * ch, ch)])
        for c in range(max(n_ch - 2, 0), n_ch):
            cps[c].wait()
            pltpu.sync_copy(bufs[c % 3],
                            o_hbm.at[pl.ds(base + c * ch, ch)])

    return k(table, idx)


def _sc_combine(opad, pos0, pos1):
    n = pos0.shape[0]
    d = opad.shape[1]
    per_w = n // _NW
    ch = 32
    n_ch = per_w // ch
    mesh = plsc.VectorSubcoreMesh(core_axis_name="core",
                                  subcore_axis_name="subcore")

    @functools.partial(
        pl.kernel,
        out_type=jax.ShapeDtypeStruct((n, d), opad.dtype),
        mesh=mesh,
        scratch_types=[
            pltpu.VMEM((per_w,), jnp.int32),
            pltpu.VMEM((per_w,), jnp.int32),
            pltpu.VMEM((ch, d), jnp.float32),
            pltpu.VMEM((ch, d), jnp.float32),
            pltpu.SemaphoreType.DMA,
            pltpu.SemaphoreType.DMA,
        ])
    def k(x_hbm, i0_hbm, i1_hbm, o_hbm, i0_v, i1_v, row0_v, row1_v,
          sem0, sem1):
        wid = lax.axis_index("subcore") * _NC + lax.axis_index("core")
        base = wid * per_w
        pltpu.sync_copy(i0_hbm.at[pl.ds(base, per_w)], i0_v)
        pltpu.sync_copy(i1_hbm.at[pl.ds(base, per_w)], i1_v)
        for c in range(n_ch):
            sl = pl.ds(c * ch, ch)
            cp0 = pltpu.async_copy(x_hbm.at[i0_v.at[sl]], row0_v, sem0)
            cp1 = pltpu.async_copy(x_hbm.at[i1_v.at[sl]], row1_v, sem1)
            cp0.wait()
            cp1.wait()

            @pl.loop(0, ch)
            def _(r):
                for q in range(d // 16):
                    qs = pl.ds(q * 16, 16)
                    row0_v[r, qs] += row1_v[r, qs]

            pltpu.sync_copy(row0_v, o_hbm.at[pl.ds(base + c * ch, ch)])

    return k(opad, pos0, pos1)


def _cast_body(w_ref, o_ref):
    o_ref[...] = w_ref[...].astype(jnp.bfloat16)


def _cast_bf16(w):
    e, a, b = w.shape
    nb = b // 1024
    return pl.pallas_call(
        _cast_body,
        grid=(e, nb),
        in_specs=[pl.BlockSpec((1, a, 1024), lambda i, j: (i, 0, j))],
        out_specs=pl.BlockSpec((1, a, 1024), lambda i, j: (i, 0, j)),
        out_shape=jax.ShapeDtypeStruct(w.shape, jnp.bfloat16),
    )(w)


def _gmm_body(be_ref, xs_ref, w1_ref, b1_ref, w2_ref, b2_ref,
              pp_ref, o_ref):
    del be_ref
    xs = xs_ref[...].astype(jnp.bfloat16)
    h = lax.dot_general(xs, w1_ref[0],
                        (((1,), (0,)), ((), ())),
                        precision=lax.Precision.DEFAULT,
                        preferred_element_type=jnp.float32)
    h = jnp.maximum(h + b1_ref[0], 0.0)
    o = lax.dot_general(h.astype(jnp.bfloat16), w2_ref[0],
                        (((1,), (0,)), ((), ())),
                        precision=lax.Precision.DEFAULT,
                        preferred_element_type=jnp.float32)
    o_ref[...] = (o + b2_ref[0]) * pp_ref[...]


def _gmm(blk_e, xs_pad, W1t, b1, W2t, b2, ppad):
    np_, d = xs_pad.shape
    e, _, h = W1t.shape
    nblk = np_ // _T
    grid_spec = pltpu.PrefetchScalarGridSpec(
        num_scalar_prefetch=1,
        grid=(nblk,),
        in_specs=[
            pl.BlockSpec((_T, d), lambda g, be: (g, 0)),
            pl.BlockSpec((1, d, h), lambda g, be: (be[g], 0, 0)),
            pl.BlockSpec((1, 1, h), lambda g, be: (be[g], 0, 0)),
            pl.BlockSpec((1, h, d), lambda g, be: (be[g], 0, 0)),
            pl.BlockSpec((1, 1, d), lambda g, be: (be[g], 0, 0)),
            pl.BlockSpec((_T, 1), lambda g, be: (g, 0)),
        ],
        out_specs=pl.BlockSpec((_T, d), lambda g, be: (g, 0)),
    )
    return pl.pallas_call(
        _gmm_body,
        grid_spec=grid_spec,
        out_shape=jax.ShapeDtypeStruct((np_, d), jnp.float32),
        compiler_params=pltpu.CompilerParams(
            dimension_semantics=("arbitrary",)),
    )(blk_e, xs_pad, W1t, b1.reshape(e, 1, h),
      W2t, b2.reshape(e, 1, d), ppad)


def kernel(x, Wg, bg, W1, b1, W2, b2):
    bx, sx, d = x.shape
    n = bx * sx
    e = Wg.shape[0]
    kk = 2
    x_flat = x.reshape(n, d)

    i1, i2, p1, p2, imp_sum, load_sum, xpack = _gate(x_flat, Wg, bg)

    e_flat = jnp.concatenate([i1, i2], axis=1).reshape(-1)
    probs_flat = jnp.concatenate([p1, p2], axis=1).reshape(-1)
    oh = (e_flat[:, None] == jnp.arange(e, dtype=jnp.int32)[None, :]
          ).astype(jnp.int32)
    cum = jnp.cumsum(oh, axis=0)
    rk = jnp.take_along_axis(cum, e_flat[:, None], axis=1)[:, 0] - 1
    counts = cum[-1]
    nblk_e = ((counts + _T - 1) // _T).astype(jnp.int32)
    cum_nblk = jnp.cumsum(nblk_e)
    bstart = jnp.concatenate(
        [jnp.zeros((1,), jnp.int32), cum_nblk[:-1]]).astype(jnp.int32)
    nblk = n * kk // _T + e
    blk_e = jnp.minimum(
        jnp.searchsorted(cum_nblk, jnp.arange(nblk, dtype=jnp.int32),
                         side="right"),
        e - 1).astype(jnp.int32)
    pp = bstart[e_flat] * _T + rk
    np_ = nblk * _T
    gidx = jnp.zeros((np_,), jnp.int32).at[pp].set(
        (jnp.arange(n * kk, dtype=jnp.int32) // kk),
        unique_indices=True)
    ppad = jnp.zeros((np_, 1), jnp.float32).at[pp, 0].set(
        probs_flat, unique_indices=True)
    pos = pp.reshape(n, kk)

    xs_pad = _sc_gather(x_flat, gidx, 32)
    w1t = jnp.swapaxes(W1, 1, 2).astype(jnp.bfloat16)
    w2t = jnp.swapaxes(W2, 1, 2).astype(jnp.bfloat16)
    opad = _gmm(blk_e, xs_pad, w1t, b1, w2t, b2, ppad)
    y = _sc_combine(opad, pos[:, 0], pos[:, 1])

    outputs = y.reshape(bx, sx, d)
    nf = jnp.float32(n)
    aux_loss = jnp.sum((imp_sum[0] / nf) * (load_sum[0] / nf)) * e
    return outputs, aux_loss


# speedup vs baseline: 1.3780x; 1.2549x over previous
import functools

import jax
import jax.numpy as jnp
from jax import lax
from jax.experimental import pallas as pl
from jax.experimental.pallas import tpu as pltpu
from jax.experimental.pallas import tpu_sc as plsc

_TB = 256
_T = 128
_LANES = 128
_NEG = -1e30


def _gate_body(x_ref, wg_ref, bg_ref, i1_ref, i2_ref, p1_ref, p2_ref,
               imp_ref, load_ref):
    g = pl.program_id(0)
    scores = jnp.dot(x_ref[...], wg_ref[...],
                     preferred_element_type=jnp.float32) + bg_ref[...]
    m = jnp.max(scores, axis=1, keepdims=True)
    ex = jnp.exp(scores - m)
    s = jnp.sum(ex, axis=1, keepdims=True)
    probs = ex / s
    iota = lax.broadcasted_iota(jnp.int32, probs.shape, 1)
    m1 = jnp.max(probs, axis=1, keepdims=True)
    a1 = jnp.min(jnp.where(probs == m1, iota, 1 << 30), axis=1, keepdims=True)
    probs_m = jnp.where(iota == a1, -1.0, probs)
    m2 = jnp.max(probs_m, axis=1, keepdims=True)
    a2 = jnp.min(jnp.where(probs_m == m2, iota, 1 << 30), axis=1,
                 keepdims=True)
    e2 = jnp.exp(m2 - m1)
    denom = 1.0 + e2
    i1_ref[...] = a1
    i2_ref[...] = a2
    p1_ref[...] = 1.0 / denom
    p2_ref[...] = e2 / denom
    imp_blk = jnp.sum(probs, axis=0, keepdims=True)
    load_blk = jnp.sum((probs > 0).astype(jnp.float32), axis=0, keepdims=True)

    @pl.when(g == 0)
    def _():
        imp_ref[...] = imp_blk
        load_ref[...] = load_blk

    @pl.when(g != 0)
    def _():
        imp_ref[...] += imp_blk
        load_ref[...] += load_blk


def _gate(x_flat, Wg, bg):
    n, d = x_flat.shape
    e = Wg.shape[0]
    wgp = jnp.zeros((d, _LANES), jnp.float32).at[:, :e].set(Wg.T)
    bgp = jnp.full((1, _LANES), _NEG, jnp.float32).at[0, :e].set(bg)
    ntb = n // _TB
    return pl.pallas_call(
        _gate_body,
        grid=(ntb,),
        in_specs=[
            pl.BlockSpec((_TB, d), lambda g: (g, 0)),
            pl.BlockSpec((d, _LANES), lambda g: (0, 0)),
            pl.BlockSpec((1, _LANES), lambda g: (0, 0)),
        ],
        out_specs=[
            pl.BlockSpec((_TB, 1), lambda g: (g, 0)),
            pl.BlockSpec((_TB, 1), lambda g: (g, 0)),
            pl.BlockSpec((_TB, 1), lambda g: (g, 0)),
            pl.BlockSpec((_TB, 1), lambda g: (g, 0)),
            pl.BlockSpec((1, _LANES), lambda g: (0, 0)),
            pl.BlockSpec((1, _LANES), lambda g: (0, 0)),
        ],
        out_shape=[
            jax.ShapeDtypeStruct((n, 1), jnp.int32),
            jax.ShapeDtypeStruct((n, 1), jnp.int32),
            jax.ShapeDtypeStruct((n, 1), jnp.float32),
            jax.ShapeDtypeStruct((n, 1), jnp.float32),
            jax.ShapeDtypeStruct((1, _LANES), jnp.float32),
            jax.ShapeDtypeStruct((1, _LANES), jnp.float32),
        ],
    )(x_flat, wgp, bgp)


_NC = 2
_NS = 16
_NW = _NC * _NS


def _sc_gather(table, idx, ch):
    np_, d = idx.shape[0], table.shape[1]
    per_w = np_ // _NW
    n_ch = per_w // ch
    mesh = plsc.VectorSubcoreMesh(core_axis_name="core",
                                  subcore_axis_name="subcore")

    @functools.partial(
        pl.kernel,
        out_type=jax.ShapeDtypeStruct((np_, d), table.dtype),
        mesh=mesh,
        scratch_types=[
            pltpu.VMEM((per_w,), jnp.int32),
            pltpu.VMEM((ch, d), table.dtype),
            pltpu.VMEM((ch, d), table.dtype),
            pltpu.VMEM((ch, d), table.dtype),
            pltpu.SemaphoreType.DMA,
            pltpu.SemaphoreType.DMA,
            pltpu.SemaphoreType.DMA,
        ])
    def k(x_hbm, i_hbm, o_hbm, idx_v, row0_v, row1_v, row2_v,
          sem0, sem1, sem2):
        wid = lax.axis_index("subcore") * _NC + lax.axis_index("core")
        base = wid * per_w
        pltpu.sync_copy(i_hbm.at[pl.ds(base, per_w)], idx_v)
        bufs = (row0_v, row1_v, row2_v)
        sems = (sem0, sem1, sem2)
        cps = []
        for c in range(n_ch):
            b = c % 3
            cp = pltpu.async_copy(
                x_hbm.at[idx_v.at[pl.ds(c * ch, ch)]], bufs[b], sems[b])
            cps.append(cp)
            if c >= 2:
                cps[c - 2].wait()
                pltpu.sync_copy(bufs[(c - 2) % 3],
                                o_hbm.at[pl.ds(base + (c - 2) * ch, ch)])
        for c in range(max(n_ch - 2, 0), n_ch):
            cps[c].wait()
            pltpu.sync_copy(bufs[c % 3],
                            o_hbm.at[pl.ds(base + c * ch, ch)])

    return k(table, idx)


def _sc_combine(opad, pos0, pos1):
    n = pos0.shape[0]
    d = opad.shape[1]
    per_w = n // _NW
    ch = 32
    n_ch = per_w // ch
    mesh = plsc.VectorSubcoreMesh(core_axis_name="core",
                                  subcore_axis_name="subcore")

    @functools.partial(
        pl.kernel,
        out_type=jax.ShapeDtypeStruct((n, d), opad.dtype),
        mesh=mesh,
        scratch_types=[
            pltpu.VMEM((per_w,), jnp.int32),
            pltpu.VMEM((per_w,), jnp.int32),
            pltpu.VMEM((ch, d), jnp.float32),
            pltpu.VMEM((ch, d), jnp.float32),
            pltpu.SemaphoreType.DMA,
            pltpu.SemaphoreType.DMA,
        ])
    def k(x_hbm, i0_hbm, i1_hbm, o_hbm, i0_v, i1_v, row0_v, row1_v,
          sem0, sem1):
        wid = lax.axis_index("subcore") * _NC + lax.axis_index("core")
        base = wid * per_w
        pltpu.sync_copy(i0_hbm.at[pl.ds(base, per_w)], i0_v)
        pltpu.sync_copy(i1_hbm.at[pl.ds(base, per_w)], i1_v)
        for c in range(n_ch):
            sl = pl.ds(c * ch, ch)
            cp0 = pltpu.async_copy(x_hbm.at[i0_v.at[sl]], row0_v, sem0)
            cp1 = pltpu.async_copy(x_hbm.at[i1_v.at[sl]], row1_v, sem1)
            cp0.wait()
            cp1.wait()

            @pl.loop(0, ch)
            def _(r):
                for q in range(d // 16):
                    qs = pl.ds(q * 16, 16)
                    row0_v[r, qs] += row1_v[r, qs]

            pltpu.sync_copy(row0_v, o_hbm.at[pl.ds(base + c * ch, ch)])

    return k(opad, pos0, pos1)


def _cast_body(w_ref, o_ref):
    o_ref[...] = w_ref[...].astype(jnp.bfloat16)


def _cast_bf16(w):
    e, a, b = w.shape
    nb = b // 1024
    return pl.pallas_call(
        _cast_body,
        grid=(e, nb),
        in_specs=[pl.BlockSpec((1, a, 1024), lambda i, j: (i, 0, j))],
        out_specs=pl.BlockSpec((1, a, 1024), lambda i, j: (i, 0, j)),
        out_shape=jax.ShapeDtypeStruct(w.shape, jnp.bfloat16),
    )(w)


def _gmm_body(be_ref, xs_ref, w1_ref, b1_ref, w2_ref, b2_ref,
              pp_ref, o_ref):
    del be_ref
    xs = xs_ref[...]
    h = lax.dot_general(xs, w1_ref[0],
                        (((1,), (1,)), ((), ())),
                        precision=lax.Precision.DEFAULT,
                        preferred_element_type=jnp.float32)
    h = jnp.maximum(h + b1_ref[0], 0.0)
    o = lax.dot_general(h, w2_ref[0],
                        (((1,), (1,)), ((), ())),
                        precision=lax.Precision.DEFAULT,
                        preferred_element_type=jnp.float32)
    o_ref[...] = (o + b2_ref[0]) * pp_ref[...]


def _gmm(blk_e, xs_pad, W1t, b1, W2t, b2, ppad):
    np_, d = xs_pad.shape
    e, h, _ = W1t.shape
    nblk = np_ // _T
    grid_spec = pltpu.PrefetchScalarGridSpec(
        num_scalar_prefetch=1,
        grid=(nblk,),
        in_specs=[
            pl.BlockSpec((_T, d), lambda g, be: (g, 0)),
            pl.BlockSpec((1, h, d), lambda g, be: (be[g], 0, 0)),
            pl.BlockSpec((1, 1, h), lambda g, be: (be[g], 0, 0)),
            pl.BlockSpec((1, d, h), lambda g, be: (be[g], 0, 0)),
            pl.BlockSpec((1, 1, d), lambda g, be: (be[g], 0, 0)),
            pl.BlockSpec((_T, 1), lambda g, be: (g, 0)),
        ],
        out_specs=pl.BlockSpec((_T, d), lambda g, be: (g, 0)),
    )
    return pl.pallas_call(
        _gmm_body,
        grid_spec=grid_spec,
        out_shape=jax.ShapeDtypeStruct((np_, d), jnp.float32),
        compiler_params=pltpu.CompilerParams(
            dimension_semantics=("arbitrary",)),
    )(blk_e, xs_pad, W1t, b1.reshape(e, 1, h),
      W2t, b2.reshape(e, 1, d), ppad)


def kernel(x, Wg, bg, W1, b1, W2, b2):
    bx, sx, d = x.shape
    n = bx * sx
    e = Wg.shape[0]
    kk = 2
    x_flat = x.reshape(n, d)

    i1, i2, p1, p2, imp_sum, load_sum = _gate(x_flat, Wg, bg)

    e_flat = jnp.concatenate([i1, i2], axis=1).reshape(-1)
    probs_flat = jnp.concatenate([p1, p2], axis=1).reshape(-1)
    oh = (e_flat[:, None] == jnp.arange(e, dtype=jnp.int32)[None, :]
          ).astype(jnp.int32)
    cum = jnp.cumsum(oh, axis=0)
    rk = jnp.take_along_axis(cum, e_flat[:, None], axis=1)[:, 0] - 1
    counts = cum[-1]
    nblk_e = ((counts + _T - 1) // _T).astype(jnp.int32)
    cum_nblk = jnp.cumsum(nblk_e)
    bstart = jnp.concatenate(
        [jnp.zeros((1,), jnp.int32), cum_nblk[:-1]]).astype(jnp.int32)
    nblk = n * kk // _T + e
    blk_e = jnp.minimum(
        jnp.searchsorted(cum_nblk, jnp.arange(nblk, dtype=jnp.int32),
                         side="right"),
        e - 1).astype(jnp.int32)
    pp = bstart[e_flat] * _T + rk
    np_ = nblk * _T
    gidx = jnp.zeros((np_,), jnp.int32).at[pp].set(
        (jnp.arange(n * kk, dtype=jnp.int32) // kk),
        unique_indices=True)
    ppad = jnp.zeros((np_, 1), jnp.float32).at[pp, 0].set(
        probs_flat, unique_indices=True)
    pos = pp.reshape(n, kk)

    xs_pad = _sc_gather(x_flat, gidx, 32)
    opad = _gmm(blk_e, xs_pad, W1, b1, W2, b2, ppad)
    y = _sc_combine(opad, pos[:, 0], pos[:, 1])

    outputs = y.reshape(bx, sx, d)
    nf = jnp.float32(n)
    aux_loss = jnp.sum((imp_sum[0] / nf) * (load_sum[0] / nf)) * e
    return outputs, aux_loss
